# baseline (speedup 1.0000x reference)
import functools

import jax
import jax.numpy as jnp
from jax import lax
from jax.experimental import pallas as pl
from jax.experimental.pallas import tpu as pltpu
from jax.experimental.pallas import tpu_sc as plsc

N_NODES = 10000
N_EDGES = 160000
D_IN = 1433
D_HID = 16
D_OUT = 7

NC = 2
NS = 16
NW = NC * NS
E_W = 5120
E_PAD = NW * E_W
N_PAD = 10240
ZROWS = N_PAD // NS


def _mm_body(x_ref, w_ref, o_ref):
    o_ref[...] = jnp.dot(x_ref[...], w_ref[...],
                         preferred_element_type=jnp.float32)


def _project(x, w):
    n, k = x.shape
    h = w.shape[1]
    br = 2000
    return pl.pallas_call(
        _mm_body,
        grid=(n // br,),
        in_specs=[
            pl.BlockSpec((br, k), lambda i: (i, 0)),
            pl.BlockSpec((k, h), lambda i: (0, 0)),
        ],
        out_specs=pl.BlockSpec((br, h), lambda i: (i, 0)),
        out_shape=jax.ShapeDtypeStruct((n, h), jnp.float32),
    )(x, w)


def _out_body(p_ref, w2_ref, b2_ref, o_ref):
    a = p_ref[0] + p_ref[1]
    logits = jnp.dot(a, w2_ref[...],
                     preferred_element_type=jnp.float32) + b2_ref[...]
    m = jnp.max(logits, axis=1, keepdims=True)
    s = logits - m
    lse = jnp.log(jnp.sum(jnp.exp(s), axis=1, keepdims=True))
    o_ref[...] = s - lse


def _final(parts, w2, b2):
    return pl.pallas_call(
        _out_body,
        out_shape=jax.ShapeDtypeStruct((N_PAD, D_OUT), jnp.float32),
    )(parts, w2, b2.reshape(1, D_OUT))


_sc_mesh = plsc.VectorSubcoreMesh(core_axis_name="c", subcore_axis_name="s")


@functools.partial(
    pl.kernel,
    mesh=_sc_mesh,
    compiler_params=pltpu.CompilerParams(use_tc_tiling_on_sc=False),
    out_type=jax.ShapeDtypeStruct((NC, N_PAD, D_HID), jnp.float32),
    scratch_types=[
        pltpu.VMEM((2, E_W), jnp.int32),
        pltpu.VMEM((2, E_W), jnp.int32),
        pltpu.VMEM((E_W, D_HID), jnp.float32),
        pltpu.VMEM((ZROWS, D_HID), jnp.float32),
        pltpu.VMEM((D_HID,), jnp.float32),
        pltpu.VMEM_SHARED((N_PAD, D_HID), jnp.float32),
        pltpu.SemaphoreType.DMA,
    ],
)
def _sc_gcn(table_hbm, src_hbm, dst_hbm, b1_hbm,
            out_hbm, src1_v, dst1_v, rows_v, zbuf_v, b1_v,
            acc_sh, sem):
    c = lax.axis_index("c")
    s = lax.axis_index("s")
    wid = c * NS + s
    mine = pl.ds(s * ZROWS, ZROWS)

    zero = jnp.zeros((D_HID,), jnp.float32)

    def _zero_row(r, carry):
        zbuf_v[r, :] = zero
        return carry

    lax.fori_loop(0, ZROWS, _zero_row, 0)
    pltpu.sync_copy(zbuf_v, acc_sh.at[mine])

    pltpu.sync_copy(src_hbm.at[pl.ds(2 * s, 2)], src1_v)
    pltpu.sync_copy(dst_hbm.at[pl.ds(2 * s, 2)], dst1_v)
    pltpu.sync_copy(b1_hbm, b1_v)
    plsc.subcore_barrier()

    for r in range(2):
        pltpu.async_copy(table_hbm.at[src1_v.at[r]], rows_v, sem).wait()
        pltpu.sync_copy(rows_v, acc_sh.at[dst1_v.at[r]], add=True)
    plsc.subcore_barrier()

    rstage = rows_v.at[pl.ds(0, ZROWS)]
    pltpu.sync_copy(acc_sh.at[mine], rstage)
    b1val = b1_v[...]

    def _relu_row(r, carry):
        rows_v[r, :] = jnp.maximum(rows_v[r, :] + b1val, 0.0)
        return carry

    lax.fori_loop(0, ZROWS, _relu_row, 0)
    pltpu.sync_copy(rstage, acc_sh.at[mine])
    plsc.subcore_barrier()

    pltpu.sync_copy(src_hbm.at[wid], src1_v.at[0])
    pltpu.sync_copy(dst_hbm.at[wid], dst1_v.at[0])
    pltpu.async_copy(acc_sh.at[src1_v.at[0]], rows_v, sem).wait()
    plsc.subcore_barrier()
    pltpu.sync_copy(zbuf_v, acc_sh.at[mine])
    plsc.subcore_barrier()
    pltpu.sync_copy(rows_v, acc_sh.at[dst1_v.at[0]], add=True)
    plsc.subcore_barrier()

    pltpu.sync_copy(acc_sh.at[mine], out_hbm.at[c, mine])


@jax.jit
def kernel(features, edge_index, W1, b1, W2, b2):
    src = edge_index[0]
    dst = edge_index[1]
    pad = E_PAD - N_EDGES
    src_p = jnp.concatenate([src, jnp.zeros((pad,), jnp.int32)])
    dst_p = jnp.concatenate([dst, jnp.full((pad,), N_NODES, jnp.int32)])
    src2 = src_p.reshape(NW, E_W)
    dst2 = dst_p.reshape(NW, E_W)

    y1 = _project(features, W1)
    p = _sc_gcn(y1, src2, dst2, b1)
    return _final(p, W2, b2)[:N_NODES]


# speedup vs baseline: 1.0360x; 1.0360x over previous
import functools

import jax
import jax.numpy as jnp
from jax import lax
from jax.experimental import pallas as pl
from jax.experimental.pallas import tpu as pltpu
from jax.experimental.pallas import tpu_sc as plsc

N_NODES = 10000
N_EDGES = 160000
D_IN = 1433
D_HID = 16
D_OUT = 7

NC = 2
NS = 16
NW = NC * NS
E_W = 5120
E_C = 2560
E_PAD = NW * E_W
N_PAD = 10240
ZROWS = N_PAD // NS


def _mm_body(x_ref, w_ref, o_ref):
    o_ref[...] = jnp.dot(x_ref[...], w_ref[...],
                         preferred_element_type=jnp.float32)


def _project(x, w):
    n, k = x.shape
    h = w.shape[1]
    br = 2000
    return pl.pallas_call(
        _mm_body,
        grid=(n // br,),
        in_specs=[
            pl.BlockSpec((br, k), lambda i: (i, 0)),
            pl.BlockSpec((k, h), lambda i: (0, 0)),
        ],
        out_specs=pl.BlockSpec((br, h), lambda i: (i, 0)),
        out_shape=jax.ShapeDtypeStruct((n, h), jnp.float32),
    )(x, w)


def _out_body(p_ref, w2_ref, b2_ref, o_ref):
    a = p_ref[0] + p_ref[1]
    logits = jnp.dot(a, w2_ref[...],
                     preferred_element_type=jnp.float32) + b2_ref[...]
    m = jnp.max(logits, axis=1, keepdims=True)
    s = logits - m
    lse = jnp.log(jnp.sum(jnp.exp(s), axis=1, keepdims=True))
    o_ref[...] = s - lse


def _final(parts, w2, b2):
    return pl.pallas_call(
        _out_body,
        out_shape=jax.ShapeDtypeStruct((N_PAD, D_OUT), jnp.float32),
    )(parts, w2, b2.reshape(1, D_OUT))


_sc_mesh = plsc.VectorSubcoreMesh(core_axis_name="c", subcore_axis_name="s")


@functools.partial(
    pl.kernel,
    mesh=_sc_mesh,
    compiler_params=pltpu.CompilerParams(use_tc_tiling_on_sc=False),
    out_type=jax.ShapeDtypeStruct((NC, N_PAD, D_HID), jnp.float32),
    scratch_types=[
        pltpu.VMEM((4, E_C), jnp.int32),
        pltpu.VMEM((4, E_C), jnp.int32),
        pltpu.VMEM((E_C, D_HID), jnp.float32),
        pltpu.VMEM((E_C, D_HID), jnp.float32),
        pltpu.VMEM((ZROWS, D_HID), jnp.float32),
        pltpu.VMEM((D_HID,), jnp.float32),
        pltpu.VMEM_SHARED((N_PAD, D_HID), jnp.float32),
        pltpu.SemaphoreType.DMA,
        pltpu.SemaphoreType.DMA,
    ],
)
def _sc_gcn(table_hbm, src_hbm, dst_hbm, b1_hbm,
            out_hbm, src_v, dst_v, rows0_v, rows1_v, zbuf_v, b1_v,
            acc_sh, sem0, sem1):
    c = lax.axis_index("c")
    s = lax.axis_index("s")
    wid = c * NS + s
    mine = pl.ds(s * ZROWS, ZROWS)

    pltpu.sync_copy(src_hbm.at[pl.ds(4 * s, 4)], src_v)
    pltpu.sync_copy(dst_hbm.at[pl.ds(4 * s, 4)], dst_v)
    pltpu.sync_copy(b1_hbm, b1_v)
    cp0 = pltpu.async_copy(table_hbm.at[src_v.at[0]], rows0_v, sem0)
    cp1 = pltpu.async_copy(table_hbm.at[src_v.at[1]], rows1_v, sem1)

    zero = jnp.zeros((D_HID,), jnp.float32)

    def _zero_row(r, carry):
        zbuf_v[r, :] = zero
        return carry

    lax.fori_loop(0, ZROWS, _zero_row, 0)
    pltpu.sync_copy(zbuf_v, acc_sh.at[mine])
    plsc.subcore_barrier()

    cp0.wait()
    pltpu.sync_copy(rows0_v, acc_sh.at[dst_v.at[0]], add=True)
    pltpu.async_copy(table_hbm.at[src_v.at[2]], rows0_v, sem0)
    cp1.wait()
    pltpu.sync_copy(rows1_v, acc_sh.at[dst_v.at[1]], add=True)
    pltpu.async_copy(table_hbm.at[src_v.at[3]], rows1_v, sem1)
    pltpu.make_async_copy(table_hbm.at[src_v.at[2]], rows0_v, sem0).wait()
    pltpu.sync_copy(rows0_v, acc_sh.at[dst_v.at[2]], add=True)
    pltpu.make_async_copy(table_hbm.at[src_v.at[3]], rows1_v, sem1).wait()
    pltpu.sync_copy(rows1_v, acc_sh.at[dst_v.at[3]], add=True)
    plsc.subcore_barrier()

    rstage = rows0_v.at[pl.ds(0, ZROWS)]
    pltpu.sync_copy(acc_sh.at[mine], rstage)
    b1val = b1_v[...]

    def _relu_row(r, carry):
        rows0_v[r, :] = jnp.maximum(rows0_v[r, :] + b1val, 0.0)
        return carry

    lax.fori_loop(0, ZROWS, _relu_row, 0)
    pltpu.sync_copy(rstage, acc_sh.at[mine])
    plsc.subcore_barrier()

    pltpu.sync_copy(src_hbm.at[pl.ds(2 * wid, 2)], src_v.at[pl.ds(0, 2)])
    pltpu.sync_copy(dst_hbm.at[pl.ds(2 * wid, 2)], dst_v.at[pl.ds(0, 2)])
    g0 = pltpu.async_copy(acc_sh.at[src_v.at[0]], rows0_v, sem0)
    g1 = pltpu.async_copy(acc_sh.at[src_v.at[1]], rows1_v, sem1)
    g0.wait()
    g1.wait()
    plsc.subcore_barrier()
    pltpu.sync_copy(zbuf_v, acc_sh.at[mine])
    plsc.subcore_barrier()
    pltpu.sync_copy(rows0_v, acc_sh.at[dst_v.at[0]], add=True)
    pltpu.sync_copy(rows1_v, acc_sh.at[dst_v.at[1]], add=True)
    plsc.subcore_barrier()

    pltpu.sync_copy(acc_sh.at[mine], out_hbm.at[c, mine])


@jax.jit
def kernel(features, edge_index, W1, b1, W2, b2):
    src = edge_index[0]
    dst = edge_index[1]
    pad = E_PAD - N_EDGES
    src_p = jnp.concatenate([src, jnp.zeros((pad,), jnp.int32)])
    dst_p = jnp.concatenate([dst, jnp.full((pad,), N_NODES, jnp.int32)])
    src2 = src_p.reshape(2 * NW, E_C)
    dst2 = dst_p.reshape(2 * NW, E_C)

    y1 = _project(features, W1)
    p = _sc_gcn(y1, src2, dst2, b1)
    return _final(p, W2, b2)[:N_NODES]


# speedup vs baseline: 1.0523x; 1.0158x over previous
import functools

import jax
import jax.numpy as jnp
from jax import lax
from jax.experimental import pallas as pl
from jax.experimental.pallas import tpu as pltpu
from jax.experimental.pallas import tpu_sc as plsc

N_NODES = 10000
N_EDGES = 160000
D_IN = 1433
D_HID = 16
D_OUT = 7

NC = 2
NS = 16
NW = NC * NS
E_W = 5120
E_C = 2560
E_PAD = NW * E_W
N_PAD = 10240
ZROWS = N_PAD // NS


def _mm_body(x_ref, w_ref, o_ref):
    o_ref[...] = jnp.dot(x_ref[...], w_ref[...],
                         preferred_element_type=jnp.float32)


def _project(x, w):
    n, k = x.shape
    br = 2000
    return pl.pallas_call(
        _mm_body,
        grid=(n // br,),
        in_specs=[
            pl.BlockSpec((br, k), lambda i: (i, 0)),
            pl.BlockSpec((k, D_HID), lambda i: (0, 0)),
        ],
        out_specs=pl.BlockSpec((br, D_HID), lambda i: (i, 0)),
        out_shape=jax.ShapeDtypeStruct((n, D_HID), jnp.float32),
    )(x, w)


def _out_body(p_ref, w2_ref, b2_ref, o_ref):
    a = p_ref[0] + p_ref[1]
    logits = jnp.dot(a, w2_ref[...],
                     preferred_element_type=jnp.float32) + b2_ref[...]
    m = jnp.max(logits, axis=1, keepdims=True)
    s = logits - m
    lse = jnp.log(jnp.sum(jnp.exp(s), axis=1, keepdims=True))
    o_ref[...] = s - lse


def _final(parts, w2, b2):
    return pl.pallas_call(
        _out_body,
        grid=(1,),
        in_specs=[
            pl.BlockSpec((NC, N_NODES, D_HID), lambda i: (0, 0, 0)),
            pl.BlockSpec((D_HID, D_OUT), lambda i: (0, 0)),
            pl.BlockSpec((1, D_OUT), lambda i: (0, 0)),
        ],
        out_specs=pl.BlockSpec((N_NODES, D_OUT), lambda i: (0, 0)),
        out_shape=jax.ShapeDtypeStruct((N_NODES, D_OUT), jnp.float32),
    )(parts, w2, b2.reshape(1, D_OUT))


_sc_mesh = plsc.VectorSubcoreMesh(core_axis_name="c", subcore_axis_name="s")


@functools.partial(
    pl.kernel,
    mesh=_sc_mesh,
    compiler_params=pltpu.CompilerParams(use_tc_tiling_on_sc=False),
    out_type=jax.ShapeDtypeStruct((NC, N_PAD, D_HID), jnp.float32),
    scratch_types=[
        pltpu.VMEM((4, E_C), jnp.int32),
        pltpu.VMEM((4, E_C), jnp.int32),
        pltpu.VMEM((E_C, D_HID), jnp.float32),
        pltpu.VMEM((E_C, D_HID), jnp.float32),
        pltpu.VMEM((ZROWS, D_HID), jnp.float32),
        pltpu.VMEM((D_HID,), jnp.float32),
        pltpu.VMEM_SHARED((N_PAD, D_HID), jnp.float32),
        pltpu.SemaphoreType.DMA,
        pltpu.SemaphoreType.DMA,
    ],
)
def _sc_gcn(table_hbm, src_hbm, dst_hbm, b1_hbm,
            out_hbm, src_v, dst_v, rows0_v, rows1_v, zbuf_v, b1_v,
            acc_sh, sem0, sem1):
    c = lax.axis_index("c")
    s = lax.axis_index("s")
    wid = c * NS + s
    mine = pl.ds(s * ZROWS, ZROWS)

    pltpu.sync_copy(src_hbm.at[pl.ds(4 * s, 4)], src_v)
    pltpu.sync_copy(dst_hbm.at[pl.ds(4 * s, 4)], dst_v)
    pltpu.sync_copy(b1_hbm, b1_v)
    cp0 = pltpu.async_copy(table_hbm.at[src_v.at[0]], rows0_v, sem0)
    cp1 = pltpu.async_copy(table_hbm.at[src_v.at[1]], rows1_v, sem1)

    zero = jnp.zeros((D_HID,), jnp.float32)

    def _zero_row(r, carry):
        for u in range(4):
            zbuf_v[r * 4 + u, :] = zero
        return carry

    lax.fori_loop(0, ZROWS // 4, _zero_row, 0)
    pltpu.sync_copy(zbuf_v, acc_sh.at[mine])
    plsc.subcore_barrier()

    cp0.wait()
    pltpu.sync_copy(rows0_v, acc_sh.at[dst_v.at[0]], add=True)
    pltpu.async_copy(table_hbm.at[src_v.at[2]], rows0_v, sem0)
    cp1.wait()
    pltpu.sync_copy(rows1_v, acc_sh.at[dst_v.at[1]], add=True)
    pltpu.async_copy(table_hbm.at[src_v.at[3]], rows1_v, sem1)
    pltpu.make_async_copy(table_hbm.at[src_v.at[2]], rows0_v, sem0).wait()
    pltpu.sync_copy(rows0_v, acc_sh.at[dst_v.at[2]], add=True)
    pltpu.make_async_copy(table_hbm.at[src_v.at[3]], rows1_v, sem1).wait()
    pltpu.sync_copy(rows1_v, acc_sh.at[dst_v.at[3]], add=True)
    plsc.subcore_barrier()

    rstage = rows0_v.at[pl.ds(0, ZROWS)]
    pltpu.sync_copy(acc_sh.at[mine], rstage)
    b1val = b1_v[...]

    def _relu_row(r, carry):
        for u in range(4):
            i = r * 4 + u
            rows0_v[i, :] = jnp.maximum(rows0_v[i, :] + b1val, 0.0)
        return carry

    lax.fori_loop(0, ZROWS // 4, _relu_row, 0)
    pltpu.sync_copy(rstage, acc_sh.at[mine])
    plsc.subcore_barrier()

    pltpu.sync_copy(src_hbm.at[pl.ds(2 * wid, 2)], src_v.at[pl.ds(0, 2)])
    pltpu.sync_copy(dst_hbm.at[pl.ds(2 * wid, 2)], dst_v.at[pl.ds(0, 2)])
    g0 = pltpu.async_copy(acc_sh.at[src_v.at[0]], rows0_v, sem0)
    g1 = pltpu.async_copy(acc_sh.at[src_v.at[1]], rows1_v, sem1)
    g0.wait()
    g1.wait()
    plsc.subcore_barrier()
    pltpu.sync_copy(zbuf_v, acc_sh.at[mine])
    plsc.subcore_barrier()
    pltpu.sync_copy(rows0_v, acc_sh.at[dst_v.at[0]], add=True)
    pltpu.sync_copy(rows1_v, acc_sh.at[dst_v.at[1]], add=True)
    plsc.subcore_barrier()

    pltpu.sync_copy(acc_sh.at[mine], out_hbm.at[c, mine])


@jax.jit
def kernel(features, edge_index, W1, b1, W2, b2):
    src = edge_index[0]
    dst = edge_index[1]
    pad = E_PAD - N_EDGES
    src_p = jnp.concatenate([src, jnp.zeros((pad,), jnp.int32)])
    dst_p = jnp.concatenate([dst, jnp.full((pad,), N_NODES, jnp.int32)])
    src2 = src_p.reshape(2 * NW, E_C)
    dst2 = dst_p.reshape(2 * NW, E_C)

    y1 = _project(features, W1)
    p = _sc_gcn(y1, src2, dst2, b1)
    return _final(p, W2, b2)
